# baseline (speedup 1.0000x reference)
import jax
import jax.numpy as jnp
from jax.experimental import pallas as pl
from jax.experimental.pallas import tpu as pltpu

_NCHUNK = 8


def _dma_copy(x_ref, o_ref, *sems):
    rows = x_ref.shape[0] // _NCHUNK
    copies = [
        pltpu.make_async_copy(
            x_ref.at[pl.ds(i * rows, rows)],
            o_ref.at[pl.ds(i * rows, rows)],
            sems[i],
        )
        for i in range(_NCHUNK)
    ]
    for c in copies:
        c.start()
    for c in copies:
        c.wait()


def kernel(x):
    return pl.pallas_call(
        _dma_copy,
        in_specs=[pl.BlockSpec(memory_space=pl.ANY)],
        out_specs=pl.BlockSpec(memory_space=pl.ANY),
        out_shape=jax.ShapeDtypeStruct(x.shape, x.dtype),
        scratch_shapes=[pltpu.SemaphoreType.DMA] * _NCHUNK,
    )(x)


# speedup vs baseline: 43.2290x; 43.2290x over previous
import jax
import jax.numpy as jnp
from jax.experimental import pallas as pl

_GRID = 32


def _copy_block(x_ref, o_ref):
    o_ref[...] = x_ref[...]


def kernel(x):
    b, c, f = x.shape
    blk = b // _GRID
    return pl.pallas_call(
        _copy_block,
        grid=(_GRID,),
        in_specs=[pl.BlockSpec((blk, c, f), lambda i: (i, 0, 0))],
        out_specs=pl.BlockSpec((blk, c, f), lambda i: (i, 0, 0)),
        out_shape=jax.ShapeDtypeStruct((b, c, f), x.dtype),
    )(x)


# speedup vs baseline: 48.9280x; 1.1318x over previous
import jax
import jax.numpy as jnp
from jax.experimental import pallas as pl

_GRID = 8


def _copy_block(x_ref, o_ref):
    o_ref[...] = x_ref[...]


def kernel(x):
    b, c, f = x.shape
    blk = b // _GRID
    return pl.pallas_call(
        _copy_block,
        grid=(_GRID,),
        in_specs=[pl.BlockSpec((blk, c, f), lambda i: (i, 0, 0))],
        out_specs=pl.BlockSpec((blk, c, f), lambda i: (i, 0, 0)),
        out_shape=jax.ShapeDtypeStruct((b, c, f), x.dtype),
    )(x)
